# baseline (speedup 1.0000x reference)
import functools

import jax
import jax.numpy as jnp
from jax import lax
from jax.experimental import pallas as pl
from jax.experimental.pallas import tpu as pltpu
from jax.experimental.pallas import tpu_sc as plsc

B = 16384
D = 128
NC = 2
NS = 16
NW = NC * NS
BPW = B // NW
CH = 128
NCH = BPW // CH


def _sc_gather_body(ent_hbm, rel_hbm, src_idx_hbm, dst_idx_hbm, out_hbm,
                    idx_v, dst_v, rows_v, sem_in, sem_out):
    wid = lax.axis_index("s") * NC + lax.axis_index("c")
    pltpu.sync_copy(src_idx_hbm.at[wid], idx_v)
    pltpu.sync_copy(dst_idx_hbm.at[wid], dst_v)
    for role in range(3):
        table = rel_hbm if role == 1 else ent_hbm
        gathers = []
        for c in range(NCH):
            gathers.append(
                pltpu.async_copy(table.at[idx_v.at[role, c]], rows_v.at[c],
                                 sem_in))
        for g in gathers:
            g.wait()
        scatters = []
        for c in range(NCH):
            scatters.append(
                pltpu.async_copy(rows_v.at[c], out_hbm.at[dst_v.at[role, c]],
                                 sem_out))
        for s in scatters:
            s.wait()


@jax.jit
def _run(entity_embedding, relation_embedding, src_idx, dst_idx):
    mesh = plsc.VectorSubcoreMesh(core_axis_name="c", subcore_axis_name="s")
    k = functools.partial(
        pl.kernel,
        out_type=jax.ShapeDtypeStruct((3 * B, D), jnp.float32),
        mesh=mesh,
        scratch_types=[
            pltpu.VMEM((3, NCH, CH), jnp.int32),
            pltpu.VMEM((3, NCH, CH), jnp.int32),
            pltpu.VMEM((NCH, CH, D), jnp.float32),
            pltpu.SemaphoreType.DMA,
            pltpu.SemaphoreType.DMA,
        ],
    )(_sc_gather_body)
    return k(entity_embedding, relation_embedding, src_idx, dst_idx)


def kernel(sample, entity_embedding, relation_embedding):
    sample = sample.astype(jnp.int32)
    src_idx = (sample.reshape(NW, BPW, 3)
               .transpose(0, 2, 1)
               .reshape(NW, 3, NCH, CH))
    rows = 3 * jnp.arange(B, dtype=jnp.int32)[None, :] + \
        jnp.arange(3, dtype=jnp.int32)[:, None]
    dst_idx = rows.reshape(3, NW, NCH, CH).transpose(1, 0, 2, 3)
    out = _run(entity_embedding, relation_embedding, src_idx, dst_idx)
    return out.reshape(B, 3, D)


# speedup vs baseline: 1.0271x; 1.0271x over previous
import functools

import jax
import jax.numpy as jnp
from jax import lax
from jax.experimental import pallas as pl
from jax.experimental.pallas import tpu as pltpu
from jax.experimental.pallas import tpu_sc as plsc

B = 16384
D = 128
NC = 2
NS = 16
NW = NC * NS
BPW = B // NW
CH = 128
NCH = BPW // CH
NTOT = 3 * NCH
NSLOT = 6


def _sc_gather_body(ent_hbm, rel_hbm, src_hbm, dst_hbm, out_hbm,
                    src_v, dst_v, rows_v,
                    sem0, sem1, sem2, sem3, sem4, sem5):
    sems = (sem0, sem1, sem2, sem3, sem4, sem5)
    wid = lax.axis_index("s") * NC + lax.axis_index("c")
    pltpu.sync_copy(src_hbm.at[wid], src_v)
    pltpu.sync_copy(dst_hbm.at[wid], dst_v)

    def gather(c, slot):
        table = rel_hbm if c // NCH == 1 else ent_hbm
        return pltpu.async_copy(table.at[src_v.at[c]], rows_v.at[slot],
                                sems[slot])

    def scatter(c, slot):
        return pltpu.async_copy(rows_v.at[slot], out_hbm.at[dst_v.at[c]],
                                sems[slot])

    g = {c: gather(c, c) for c in range(NSLOT)}
    s = {}
    for c in range(NTOT):
        slot = c % NSLOT
        g[c].wait()
        s[c] = scatter(c, slot)
        n = c + NSLOT
        if n < NTOT:
            s[c].wait()
            g[n] = gather(n, slot)
    for c in range(NTOT - NSLOT, NTOT):
        s[c].wait()


@jax.jit
def _run(entity_embedding, relation_embedding, src_idx, dst_idx):
    mesh = plsc.VectorSubcoreMesh(core_axis_name="c", subcore_axis_name="s")
    k = functools.partial(
        pl.kernel,
        out_type=jax.ShapeDtypeStruct((3 * B, D), jnp.float32),
        mesh=mesh,
        scratch_types=[
            pltpu.VMEM((NTOT, CH), jnp.int32),
            pltpu.VMEM((NTOT, CH), jnp.int32),
            pltpu.VMEM((NSLOT, CH, D), jnp.float32),
            pltpu.SemaphoreType.DMA,
            pltpu.SemaphoreType.DMA,
            pltpu.SemaphoreType.DMA,
            pltpu.SemaphoreType.DMA,
            pltpu.SemaphoreType.DMA,
            pltpu.SemaphoreType.DMA,
        ],
    )(_sc_gather_body)
    return k(entity_embedding, relation_embedding, src_idx, dst_idx)


def kernel(sample, entity_embedding, relation_embedding):
    sample = sample.astype(jnp.int32)
    src_idx = (sample.reshape(NW, NCH, CH, 3)
               .transpose(0, 3, 1, 2)
               .reshape(NW, NTOT, CH))
    b_idx = jnp.arange(B, dtype=jnp.int32).reshape(NW, NCH, CH)
    dst_idx = (3 * b_idx[:, None, :, :] +
               jnp.arange(3, dtype=jnp.int32)[None, :, None, None])
    dst_idx = dst_idx.reshape(NW, NTOT, CH)
    out = _run(entity_embedding, relation_embedding, src_idx, dst_idx)
    return out.reshape(B, 3, D)


# speedup vs baseline: 1.7413x; 1.6953x over previous
import functools

import jax
import jax.numpy as jnp
from jax import lax
from jax.experimental import pallas as pl
from jax.experimental.pallas import tpu as pltpu
from jax.experimental.pallas import tpu_sc as plsc

B = 16384
D = 128
NC = 2
NS = 16
NW = NC * NS
BPW = B // NW
CH = 128
NCH = BPW // CH
NTOT = 3 * NCH
NSLOT = 6


def _sc_gather_body(ent_hbm, rel_hbm, src_hbm, out_hbm, src_v, rows_v,
                    sem0, sem1, sem2, sem3, sem4, sem5):
    sems = (sem0, sem1, sem2, sem3, sem4, sem5)
    wid = lax.axis_index("s") * NC + lax.axis_index("c")
    base = wid * BPW
    pltpu.sync_copy(src_hbm.at[wid], src_v)

    def gather(c, slot):
        table = rel_hbm if c // NCH == 1 else ent_hbm
        return pltpu.async_copy(table.at[src_v.at[c]],
                                rows_v.at[slot, :, 0, :], sems[slot])

    def scatter(c, slot):
        role = c // NCH
        chunk = c % NCH
        return pltpu.async_copy(
            rows_v.at[slot],
            out_hbm.at[pl.ds(base + chunk * CH, CH), pl.ds(role, 1)],
            sems[slot])

    g = {c: gather(c, c) for c in range(NSLOT)}
    s = {}
    for c in range(NTOT):
        slot = c % NSLOT
        g[c].wait()
        s[c] = scatter(c, slot)
        n = c + NSLOT
        if n < NTOT:
            s[c].wait()
            g[n] = gather(n, slot)
    for c in range(NTOT - NSLOT, NTOT):
        s[c].wait()


@jax.jit
def _run(entity_embedding, relation_embedding, src_idx):
    mesh = plsc.VectorSubcoreMesh(core_axis_name="c", subcore_axis_name="s")
    k = functools.partial(
        pl.kernel,
        out_type=jax.ShapeDtypeStruct((B, 3, D), jnp.float32),
        mesh=mesh,
        scratch_types=[
            pltpu.VMEM((NTOT, CH), jnp.int32),
            pltpu.VMEM((NSLOT, CH, 1, D), jnp.float32),
            pltpu.SemaphoreType.DMA,
            pltpu.SemaphoreType.DMA,
            pltpu.SemaphoreType.DMA,
            pltpu.SemaphoreType.DMA,
            pltpu.SemaphoreType.DMA,
            pltpu.SemaphoreType.DMA,
        ],
    )(_sc_gather_body)
    return k(entity_embedding, relation_embedding, src_idx)


def kernel(sample, entity_embedding, relation_embedding):
    sample = sample.astype(jnp.int32)
    src_idx = (sample.reshape(NW, NCH, CH, 3)
               .transpose(0, 3, 1, 2)
               .reshape(NW, NTOT, CH))
    return _run(entity_embedding, relation_embedding, src_idx)


# speedup vs baseline: 2.7071x; 1.5546x over previous
import functools

import jax
import jax.numpy as jnp
from jax import lax
from jax.experimental import pallas as pl
from jax.experimental.pallas import tpu as pltpu
from jax.experimental.pallas import tpu_sc as plsc

B = 16384
D = 128
NC = 2
NS = 16
NW = NC * NS
BPW = B // NW
CH = 128
NCH = BPW // CH
NTOT = 3 * NCH
NSLOT = 6


def _sc_gather_body(ent_hbm, rel_hbm, src_hbm, out_hbm, src_v, rows_v,
                    sem0, sem1, sem2, sem3, sem4, sem5):
    sems = (sem0, sem1, sem2, sem3, sem4, sem5)
    wid = lax.axis_index("s") * NC + lax.axis_index("c")
    base = wid * BPW
    pltpu.sync_copy(src_hbm.at[wid], src_v)

    def gather(c, slot):
        table = rel_hbm if c // NCH == 1 else ent_hbm
        return pltpu.async_copy(table.at[src_v.at[c]], rows_v.at[slot],
                                sems[slot])

    def scatter(c, slot):
        role = c // NCH
        chunk = c % NCH
        return pltpu.async_copy(
            rows_v.at[slot],
            out_hbm.at[role, pl.ds(base + chunk * CH, CH)],
            sems[slot])

    g = {c: gather(c, c) for c in range(NSLOT)}
    s = {}
    for c in range(NTOT):
        slot = c % NSLOT
        g[c].wait()
        s[c] = scatter(c, slot)
        n = c + NSLOT
        if n < NTOT:
            s[c].wait()
            g[n] = gather(n, slot)
    for c in range(NTOT - NSLOT, NTOT):
        s[c].wait()


@jax.jit
def _run(entity_embedding, relation_embedding, src_idx):
    mesh = plsc.VectorSubcoreMesh(core_axis_name="c", subcore_axis_name="s")
    k = functools.partial(
        pl.kernel,
        out_type=jax.ShapeDtypeStruct((3, B, D), jnp.float32),
        mesh=mesh,
        scratch_types=[
            pltpu.VMEM((NTOT, CH), jnp.int32),
            pltpu.VMEM((NSLOT, CH, D), jnp.float32),
            pltpu.SemaphoreType.DMA,
            pltpu.SemaphoreType.DMA,
            pltpu.SemaphoreType.DMA,
            pltpu.SemaphoreType.DMA,
            pltpu.SemaphoreType.DMA,
            pltpu.SemaphoreType.DMA,
        ],
    )(_sc_gather_body)
    out3 = k(entity_embedding, relation_embedding, src_idx)
    return out3.transpose(1, 0, 2)


def kernel(sample, entity_embedding, relation_embedding):
    sample = sample.astype(jnp.int32)
    src_idx = (sample.reshape(NW, NCH, CH, 3)
               .transpose(0, 3, 1, 2)
               .reshape(NW, NTOT, CH))
    return _run(entity_embedding, relation_embedding, src_idx)


# speedup vs baseline: 2.7187x; 1.0043x over previous
import functools

import jax
import jax.numpy as jnp
from jax import lax
from jax.experimental import pallas as pl
from jax.experimental.pallas import tpu as pltpu
from jax.experimental.pallas import tpu_sc as plsc

B = 16384
D = 128
NC = 2
NS = 16
NW = NC * NS
BPW = B // NW
CH = 128
NCH = BPW // CH
NTOT = 3 * NCH
NSLOT = 6


def _sc_gather_body(ent_hbm, rel_hbm, src_hbm, out_hbm, src_v, rows_v,
                    sem0, sem1, sem2, sem3, sem4, sem5):
    sems = (sem0, sem1, sem2, sem3, sem4, sem5)
    wid = lax.axis_index("s") * NC + lax.axis_index("c")
    base = wid * BPW
    pltpu.sync_copy(src_hbm.at[:, pl.ds(wid * NCH, NCH)], src_v)

    def gather(c, slot):
        table = rel_hbm if c // NCH == 1 else ent_hbm
        return pltpu.async_copy(table.at[src_v.at[c // NCH, c % NCH]],
                                rows_v.at[slot], sems[slot])

    def scatter(c, slot):
        role = c // NCH
        chunk = c % NCH
        return pltpu.async_copy(
            rows_v.at[slot],
            out_hbm.at[role, pl.ds(base + chunk * CH, CH)],
            sems[slot])

    g = {c: gather(c, c) for c in range(NSLOT)}
    s = {}
    for c in range(NTOT):
        slot = c % NSLOT
        g[c].wait()
        s[c] = scatter(c, slot)
        n = c + NSLOT
        if n < NTOT:
            s[c].wait()
            g[n] = gather(n, slot)
    for c in range(NTOT - NSLOT, NTOT):
        s[c].wait()


@jax.jit
def _run(entity_embedding, relation_embedding, src_idx):
    mesh = plsc.VectorSubcoreMesh(core_axis_name="c", subcore_axis_name="s")
    k = functools.partial(
        pl.kernel,
        out_type=jax.ShapeDtypeStruct((3, B, D), jnp.float32),
        mesh=mesh,
        scratch_types=[
            pltpu.VMEM((3, NCH, CH), jnp.int32),
            pltpu.VMEM((NSLOT, CH, D), jnp.float32),
            pltpu.SemaphoreType.DMA,
            pltpu.SemaphoreType.DMA,
            pltpu.SemaphoreType.DMA,
            pltpu.SemaphoreType.DMA,
            pltpu.SemaphoreType.DMA,
            pltpu.SemaphoreType.DMA,
        ],
    )(_sc_gather_body)
    out3 = k(entity_embedding, relation_embedding, src_idx)
    return out3.transpose(1, 0, 2)


def kernel(sample, entity_embedding, relation_embedding):
    src_idx = sample.astype(jnp.int32).T.reshape(3, B // CH, CH)
    return _run(entity_embedding, relation_embedding, src_idx)


# speedup vs baseline: 2.7317x; 1.0048x over previous
import functools

import jax
import jax.numpy as jnp
from jax import lax
from jax.experimental import pallas as pl
from jax.experimental.pallas import tpu as pltpu
from jax.experimental.pallas import tpu_sc as plsc

B = 16384
D = 128
NC = 2
NS = 16
NW = NC * NS
BPW = B // NW
CH = 128
NCH = BPW // CH
NTOT = 3 * NCH
NSLOT = 7


def _sc_gather_body(ent_hbm, rel_hbm, src_hbm, out_hbm, src_v, rows_v,
                    sem0, sem1, sem2, sem3, sem4, sem5, sem6):
    sems = (sem0, sem1, sem2, sem3, sem4, sem5, sem6)
    wid = lax.axis_index("s") * NC + lax.axis_index("c")
    base = wid * BPW
    pltpu.sync_copy(src_hbm.at[:, pl.ds(wid * NCH, NCH)], src_v)

    def gather(c, slot):
        table = rel_hbm if c // NCH == 1 else ent_hbm
        return pltpu.async_copy(table.at[src_v.at[c // NCH, c % NCH]],
                                rows_v.at[slot], sems[slot])

    def scatter(c, slot):
        role = c // NCH
        chunk = c % NCH
        return pltpu.async_copy(
            rows_v.at[slot],
            out_hbm.at[role, pl.ds(base + chunk * CH, CH)],
            sems[slot])

    g = {c: gather(c, c) for c in range(NSLOT)}
    s = {}
    for c in range(NTOT):
        slot = c % NSLOT
        g[c].wait()
        s[c] = scatter(c, slot)
        n = c + NSLOT
        if n < NTOT:
            s[c].wait()
            g[n] = gather(n, slot)
    for c in range(NTOT - NSLOT, NTOT):
        s[c].wait()


@jax.jit
def _run(entity_embedding, relation_embedding, src_idx):
    mesh = plsc.VectorSubcoreMesh(core_axis_name="c", subcore_axis_name="s")
    k = functools.partial(
        pl.kernel,
        out_type=jax.ShapeDtypeStruct((3, B, D), jnp.float32),
        mesh=mesh,
        scratch_types=[
            pltpu.VMEM((3, NCH, CH), jnp.int32),
            pltpu.VMEM((NSLOT, CH, D), jnp.float32),
            pltpu.SemaphoreType.DMA,
            pltpu.SemaphoreType.DMA,
            pltpu.SemaphoreType.DMA,
            pltpu.SemaphoreType.DMA,
            pltpu.SemaphoreType.DMA,
            pltpu.SemaphoreType.DMA,
            pltpu.SemaphoreType.DMA,
        ],
    )(_sc_gather_body)
    out3 = k(entity_embedding, relation_embedding, src_idx)
    return out3.transpose(1, 0, 2)


def kernel(sample, entity_embedding, relation_embedding):
    src_idx = sample.astype(jnp.int32).T.reshape(3, B // CH, CH)
    return _run(entity_embedding, relation_embedding, src_idx)
